# baseline (speedup 1.0000x reference)
import functools

import jax
import jax.numpy as jnp
from jax import lax
from jax.experimental import pallas as pl
from jax.experimental.pallas import tpu as pltpu
from jax.experimental.pallas import tpu_sc as plsc

B = 16384
V = 100001
VP = 100032
D = 64
L = 16

_info = plsc.get_sparse_core_info()
NC, NS = _info.num_cores, _info.num_subcores
FPT = D // NS
SEG = 2048
NSEG = B // SEG
UNROLL = 4

_mesh = plsc.VectorSubcoreMesh(core_axis_name="c", subcore_axis_name="s")


@functools.partial(
    pl.kernel,
    out_type=jax.ShapeDtypeStruct((2 * D, B), jnp.float32),
    mesh=_mesh,
    compiler_params=pltpu.CompilerParams(
        use_tc_tiling_on_sc=False, needs_layout_passes=False),
    scratch_types=[
        pltpu.VMEM((VP,), jnp.float32),
        pltpu.VMEM((NSEG, SEG), jnp.int32),
        pltpu.VMEM((SEG,), jnp.float32),
        pltpu.VMEM((SEG,), jnp.float32),
        pltpu.SemaphoreType.DMA,
        pltpu.SemaphoreType.DMA,
    ],
)
def _encode(yt_hbm, w0t_hbm, w1t_hbm, out_hbm,
            row_v, idx_v, ga, gb, sem_a, sem_b):
    cid = lax.axis_index("c")
    sid = lax.axis_index("s")

    pltpu.sync_copy(yt_hbm.at[pl.ds(cid * NSEG, NSEG)], idx_v)

    gbufs = (ga, gb)
    sems = (sem_a, sem_b)
    flushes = [None, None]
    for k in range(FPT):
        d = sid * FPT + k

        @pl.when(cid == 0)
        def _():
            pltpu.sync_copy(w0t_hbm.at[d], row_v)

        @pl.when(cid != 0)
        def _():
            pltpu.sync_copy(w1t_hbm.at[d], row_v)

        for m in range(NSEG):
            slot = (k * NSEG + m) % 2
            buf = gbufs[slot]
            if flushes[slot] is not None:
                flushes[slot].wait()

            def _seg_body(mm, _):
                for u in range(UNROLL):
                    o = (mm * UNROLL + u) * L
                    i16 = idx_v[m, pl.ds(o, L)]
                    buf[pl.ds(o, L)] = plsc.load_gather(row_v, [i16])
                return 0

            lax.fori_loop(0, SEG // (L * UNROLL), _seg_body, 0)
            flushes[slot] = pltpu.async_copy(
                buf,
                out_hbm.at[cid * D + d, pl.ds(m * SEG, SEG)],
                sems[slot])
    for f in flushes:
        if f is not None:
            f.wait()


def kernel(y, W0, W1):
    yt = y.astype(jnp.int32).T.reshape(2 * NSEG, SEG)
    w0p = jnp.pad(W0.T, ((0, 0), (0, VP - V)))
    w1p = jnp.pad(W1.T, ((0, 0), (0, VP - V)))
    out_fm = _encode(yt, w0p, w1p)
    return out_fm.T


# speedup vs baseline: 1.4554x; 1.4554x over previous
import functools

import jax
import jax.numpy as jnp
from jax import lax
from jax.experimental import pallas as pl
from jax.experimental.pallas import tpu as pltpu
from jax.experimental.pallas import tpu_sc as plsc

B = 16384
D = 64

_info = plsc.get_sparse_core_info()
NC, NS = _info.num_cores, _info.num_subcores
NW = NC * NS
BPW = B // NW
CHUNK = 128
NCHUNK = BPW // CHUNK
NBUF = 4

_mesh = plsc.VectorSubcoreMesh(core_axis_name="c", subcore_axis_name="s")


@functools.partial(
    pl.kernel,
    out_type=jax.ShapeDtypeStruct((B, 2 * D), jnp.float32),
    mesh=_mesh,
    compiler_params=pltpu.CompilerParams(use_tc_tiling_on_sc=False),
    scratch_types=[
        pltpu.VMEM((NCHUNK, CHUNK), jnp.int32),
        pltpu.VMEM((NCHUNK, CHUNK), jnp.int32),
    ]
    + [pltpu.VMEM((CHUNK, 2 * D), jnp.float32) for _ in range(NBUF)]
    + [pltpu.SemaphoreType.DMA for _ in range(NBUF)]
    + [pltpu.SemaphoreType.DMA],
)
def _encode(yt_hbm, w_hbm, out_hbm,
            idx0_v, idx1_v, b0, b1, b2, b3, s0, s1, s2, s3, wsem):
    wid = lax.axis_index("s") * NC + lax.axis_index("c")
    base = wid * BPW
    bufs = (b0, b1, b2, b3)
    sems = (s0, s1, s2, s3)

    pltpu.sync_copy(yt_hbm.at[0, pl.ds(wid * NCHUNK, NCHUNK)], idx0_v)
    pltpu.sync_copy(yt_hbm.at[1, pl.ds(wid * NCHUNK, NCHUNK)], idx1_v)

    gathers = [
        pltpu.async_copy(w_hbm.at[idx0_v.at[j]], bufs[j], sems[j])
        for j in range(NCHUNK)
    ]
    writes = []
    for j in range(NCHUNK):
        gathers[j].wait()
        writes.append(pltpu.async_copy(
            bufs[j], out_hbm.at[pl.ds(base + j * CHUNK, CHUNK)], wsem))
    gathers1 = []
    for j in range(NCHUNK):
        writes[j].wait()
        gathers1.append(
            pltpu.async_copy(w_hbm.at[idx1_v.at[j]], bufs[j], sems[j]))
    writes1 = []
    for j in range(NCHUNK):
        gathers1[j].wait()
        writes1.append(pltpu.async_copy(
            bufs[j].at[:, pl.ds(D, D)],
            out_hbm.at[pl.ds(base + j * CHUNK, CHUNK), pl.ds(D, D)],
            wsem))
    for w in writes1:
        w.wait()


def kernel(y, W0, W1):
    yt = y.astype(jnp.int32).T.reshape(2, NW * NCHUNK, CHUNK)
    w = jnp.concatenate([W0, W1], axis=1)
    return _encode(yt, w)


# speedup vs baseline: 1.4666x; 1.0077x over previous
import functools

import jax
import jax.numpy as jnp
from jax import lax
from jax.experimental import pallas as pl
from jax.experimental.pallas import tpu as pltpu
from jax.experimental.pallas import tpu_sc as plsc

B = 16384
D = 64

_info = plsc.get_sparse_core_info()
NC, NS = _info.num_cores, _info.num_subcores
NW = NC * NS
BPW = B // NW
CHUNK = 128
NCHUNK = BPW // CHUNK
NBUF = 6

_mesh = plsc.VectorSubcoreMesh(core_axis_name="c", subcore_axis_name="s")


@functools.partial(
    pl.kernel,
    out_type=jax.ShapeDtypeStruct((B, 2 * D), jnp.float32),
    mesh=_mesh,
    compiler_params=pltpu.CompilerParams(use_tc_tiling_on_sc=False),
    scratch_types=[
        pltpu.VMEM((NCHUNK, CHUNK), jnp.int32),
        pltpu.VMEM((NCHUNK, CHUNK), jnp.int32),
    ]
    + [pltpu.VMEM((CHUNK, 2 * D), jnp.float32) for _ in range(NBUF)]
    + [pltpu.SemaphoreType.DMA for _ in range(NBUF)]
    + [pltpu.SemaphoreType.DMA],
)
def _encode(yt_hbm, w_hbm, out_hbm,
            idx0_v, idx1_v, b0, b1, b2, b3, b4, b5,
            s0, s1, s2, s3, s4, s5, wsem):
    wid = lax.axis_index("s") * NC + lax.axis_index("c")
    base = wid * BPW
    bufs = (b0, b1, b2, b3, b4, b5)
    sems = (s0, s1, s2, s3, s4, s5)

    pltpu.sync_copy(yt_hbm.at[0, pl.ds(wid * NCHUNK, NCHUNK)], idx0_v)
    pltpu.sync_copy(yt_hbm.at[1, pl.ds(wid * NCHUNK, NCHUNK)], idx1_v)

    g0 = [
        pltpu.async_copy(w_hbm.at[idx0_v.at[j]], bufs[j], sems[j])
        for j in range(NCHUNK)
    ]
    g1 = [
        pltpu.async_copy(w_hbm.at[idx1_v.at[j]], bufs[NCHUNK + j],
                         sems[NCHUNK + j])
        for j in range(2)
    ]
    w0 = []
    for j in range(NCHUNK):
        g0[j].wait()
        w0.append(pltpu.async_copy(
            bufs[j], out_hbm.at[pl.ds(base + j * CHUNK, CHUNK)], wsem))
    for j in range(2):
        w0[j].wait()
        g1.append(
            pltpu.async_copy(w_hbm.at[idx1_v.at[2 + j]], bufs[j], sems[j]))
    g1_buf = (b4, b5, b0, b1)
    w1 = []
    for j in range(NCHUNK):
        g1[j].wait()
        w1.append(pltpu.async_copy(
            g1_buf[j].at[:, pl.ds(D, D)],
            out_hbm.at[pl.ds(base + j * CHUNK, CHUNK), pl.ds(D, D)],
            wsem))
    for w in w1:
        w.wait()
    w0[2].wait()
    w0[3].wait()


def kernel(y, W0, W1):
    yt = y.astype(jnp.int32).T.reshape(2, NW * NCHUNK, CHUNK)
    w = jnp.concatenate([W0, W1], axis=1)
    return _encode(yt, w)
